# Initial kernel scaffold; baseline (speedup 1.0000x reference)
import jax
import jax.numpy as jnp
from jax.experimental import pallas as pl


def kernel(init_state, A, loc):
    raise NotImplementedError("write your pallas kernel here")


# speedup vs baseline: 7.2365x; 7.2365x over previous
import jax
import jax.numpy as jnp
import numpy as np
from jax.experimental import pallas as pl
from jax.experimental.pallas import tpu as pltpu

_N_CHAINS = 32
_DIM = 2048
_N_STEPS = 100
_BETA_MIN = 0.05
_STEP_SIZE = 0.02


def _pt_kernel(noise_ref, lu_ref, init_ref, A_ref, loc_ref, betas_ref,
               out_ref, state_s, ll_s):
    t = pl.program_id(0)

    A = A_ref[...]
    loc = loc_ref[...]
    betas = betas_ref[...]

    @pl.when(t == 0)
    def _init():
        st0 = init_ref[...]
        y0 = jnp.dot(st0 - loc, A, preferred_element_type=jnp.float32)
        state_s[...] = st0
        ll_s[...] = -0.5 * jnp.sum(y0 * y0, axis=1, keepdims=True)

    state = state_s[...]
    ll = ll_s[...]
    noise = noise_ref[0]
    lu = lu_ref[0]
    log_u = lu[:, 0:1]
    log_u2 = lu[:, 1:2]

    prop = state + _STEP_SIZE * noise
    y = jnp.dot(prop - loc, A, preferred_element_type=jnp.float32)
    ll_prop = -0.5 * jnp.sum(y * y, axis=1, keepdims=True)
    accept = log_u < betas * (ll_prop - ll)
    state = jnp.where(accept, prop, state)
    ll = jnp.where(accept, ll_prop, ll)

    parity = t % 2
    row = jax.lax.broadcasted_iota(jnp.int32, (_N_CHAINS, 1), 0)
    is_up = (row % 2) == parity
    st_up = jnp.roll(state, -1, axis=0)
    st_dn = jnp.roll(state, 1, axis=0)
    partner_state = jnp.where(is_up, st_up, st_dn)
    partner_ll = jnp.where(is_up, jnp.roll(ll, -1, axis=0),
                           jnp.roll(ll, 1, axis=0))
    partner_beta = jnp.where(is_up, jnp.roll(betas, -1, axis=0),
                             jnp.roll(betas, 1, axis=0))
    delta = (betas - partner_beta) * (partner_ll - ll)
    pair_u = jnp.where(is_up, log_u2, jnp.roll(log_u2, 1, axis=0))
    edge = (row == 0) | (row == _N_CHAINS - 1)
    valid = (parity == 0) | jnp.logical_not(edge)
    do_swap = (pair_u < delta) & valid
    state = jnp.where(do_swap, partner_state, state)
    ll = jnp.where(do_swap, partner_ll, ll)

    state_s[...] = state
    ll_s[...] = ll

    @pl.when(t == _N_STEPS - 1)
    def _fin():
        out_ref[...] = state


def _make_pallas_call(interpret=False):
    return pl.pallas_call(
        _pt_kernel,
        grid=(_N_STEPS,),
        in_specs=[
            pl.BlockSpec((1, _N_CHAINS, _DIM), lambda t: (t, 0, 0)),
            pl.BlockSpec((1, _N_CHAINS, 2), lambda t: (t, 0, 0)),
            pl.BlockSpec((_N_CHAINS, _DIM), lambda t: (0, 0)),
            pl.BlockSpec((_DIM, _DIM), lambda t: (0, 0)),
            pl.BlockSpec((1, _DIM), lambda t: (0, 0)),
            pl.BlockSpec((_N_CHAINS, 1), lambda t: (0, 0)),
        ],
        out_specs=pl.BlockSpec((_N_CHAINS, _DIM), lambda t: (0, 0)),
        out_shape=jax.ShapeDtypeStruct((_N_CHAINS, _DIM), jnp.float32),
        scratch_shapes=[
            pltpu.VMEM((_N_CHAINS, _DIM), jnp.float32),
            pltpu.VMEM((_N_CHAINS, 1), jnp.float32),
        ],
        interpret=interpret,
    )


def kernel(init_state, A, loc):
    base_key = jax.random.key(42)

    def draws(t):
        key = jax.random.fold_in(base_key, t)
        k1, k2, k3 = jax.random.split(key, 3)
        noise = jax.random.normal(k1, (_N_CHAINS, _DIM), dtype=jnp.float32)
        log_u = jnp.log(jax.random.uniform(k2, (_N_CHAINS,), minval=1e-12))
        log_u2 = jnp.log(jax.random.uniform(k3, (_N_CHAINS,), minval=1e-12))
        return noise, log_u, log_u2

    noise_all, log_u_all, log_u2_all = jax.vmap(draws)(jnp.arange(_N_STEPS))
    lu = jnp.stack([log_u_all, log_u2_all], axis=-1)
    betas = jnp.asarray(
        _BETA_MIN ** (np.arange(_N_CHAINS) / (_N_CHAINS - 1)),
        dtype=jnp.float32).reshape(_N_CHAINS, 1)
    loc2 = loc.reshape(1, _DIM)

    return _make_pallas_call()(noise_all, lu, init_state, A, loc2, betas)


# speedup vs baseline: 7.5668x; 1.0456x over previous
import jax
import jax.numpy as jnp
import numpy as np
from jax.experimental import pallas as pl
from jax.experimental.pallas import tpu as pltpu

_N_CHAINS = 32
_DIM = 2048
_N_STEPS = 100
_BETA_MIN = 0.05
_STEP_SIZE = 0.02


def _threefry_normals(k1, k2):
    u32 = jnp.uint32
    rows = jax.lax.broadcasted_iota(u32, (_N_CHAINS, _DIM), 0)
    cols = jax.lax.broadcasted_iota(u32, (_N_CHAINS, _DIM), 1)
    counts_lo = rows * u32(_DIM) + cols
    ks0 = k1
    ks1 = k2
    ks2 = k1 ^ k2 ^ u32(0x1BD11BDA)

    x_a = jnp.zeros((_N_CHAINS, _DIM), u32) + ks0
    x_b = counts_lo + ks1

    def rot(x, r):
        return (x << u32(r)) | (x >> u32(32 - r))

    def round4(xa, xb, rots):
        for r in rots:
            xa = xa + xb
            xb = rot(xb, r)
            xb = xb ^ xa
        return xa, xb

    rot_a = (13, 15, 26, 6)
    rot_b = (17, 29, 16, 24)
    xa, xb = round4(x_a, x_b, rot_a)
    xa, xb = xa + ks1, xb + ks2 + u32(1)
    xa, xb = round4(xa, xb, rot_b)
    xa, xb = xa + ks2, xb + ks0 + u32(2)
    xa, xb = round4(xa, xb, rot_a)
    xa, xb = xa + ks0, xb + ks1 + u32(3)
    xa, xb = round4(xa, xb, rot_b)
    xa, xb = xa + ks1, xb + ks2 + u32(4)
    xa, xb = round4(xa, xb, rot_a)
    xa, xb = xa + ks2, xb + ks0 + u32(5)

    bits = xa ^ xb
    float_bits = (bits >> u32(9)) | u32(0x3F800000)
    floats = jax.lax.bitcast_convert_type(float_bits, jnp.float32) - jnp.float32(1.0)
    lo = np.nextafter(np.float32(-1.0), np.float32(0.0), dtype=np.float32)
    span = np.float32(np.float32(1.0) - lo)
    u = jnp.maximum(jnp.float32(lo), floats * jnp.float32(span) + jnp.float32(lo))
    return jnp.float32(np.sqrt(2).astype(np.float32)) * jax.lax.erf_inv(u)


def _pt_kernel(keys_ref, lu_ref, init_ref, A_ref, loc_ref, betas_ref,
               out_ref, state_s, ll_s):
    t = pl.program_id(0)

    A = A_ref[...]
    loc = loc_ref[...]
    betas = betas_ref[...]

    @pl.when(t == 0)
    def _init():
        st0 = init_ref[...]
        y0 = jnp.dot(st0 - loc, A, preferred_element_type=jnp.float32)
        state_s[...] = st0
        ll_s[...] = -0.5 * jnp.sum(y0 * y0, axis=1, keepdims=True)

    state = state_s[...]
    ll = ll_s[...]
    noise = _threefry_normals(keys_ref[0, 0, 0], keys_ref[0, 0, 1])
    lu = lu_ref[0]
    log_u = lu[:, 0:1]
    log_u2 = lu[:, 1:2]

    prop = state + _STEP_SIZE * noise
    y = jnp.dot(prop - loc, A, preferred_element_type=jnp.float32)
    ll_prop = -0.5 * jnp.sum(y * y, axis=1, keepdims=True)
    accept = log_u < betas * (ll_prop - ll)
    state = jnp.where(accept, prop, state)
    ll = jnp.where(accept, ll_prop, ll)

    parity = t % 2
    row = jax.lax.broadcasted_iota(jnp.int32, (_N_CHAINS, 1), 0)
    is_up = (row % 2) == parity
    st_up = jnp.roll(state, -1, axis=0)
    st_dn = jnp.roll(state, 1, axis=0)
    partner_state = jnp.where(is_up, st_up, st_dn)
    partner_ll = jnp.where(is_up, jnp.roll(ll, -1, axis=0),
                           jnp.roll(ll, 1, axis=0))
    partner_beta = jnp.where(is_up, jnp.roll(betas, -1, axis=0),
                             jnp.roll(betas, 1, axis=0))
    delta = (betas - partner_beta) * (partner_ll - ll)
    pair_u = jnp.where(is_up, log_u2, jnp.roll(log_u2, 1, axis=0))
    edge = (row == 0) | (row == _N_CHAINS - 1)
    valid = (parity == 0) | jnp.logical_not(edge)
    do_swap = (pair_u < delta) & valid
    state = jnp.where(do_swap, partner_state, state)
    ll = jnp.where(do_swap, partner_ll, ll)

    state_s[...] = state
    ll_s[...] = ll

    @pl.when(t == _N_STEPS - 1)
    def _fin():
        out_ref[...] = state


def _make_pallas_call(interpret=False):
    return pl.pallas_call(
        _pt_kernel,
        grid=(_N_STEPS,),
        in_specs=[
            pl.BlockSpec((1, 1, 2), lambda t: (t, 0, 0),
                         memory_space=pltpu.SMEM),
            pl.BlockSpec((1, _N_CHAINS, 2), lambda t: (t, 0, 0)),
            pl.BlockSpec((_N_CHAINS, _DIM), lambda t: (0, 0)),
            pl.BlockSpec((_DIM, _DIM), lambda t: (0, 0)),
            pl.BlockSpec((1, _DIM), lambda t: (0, 0)),
            pl.BlockSpec((_N_CHAINS, 1), lambda t: (0, 0)),
        ],
        out_specs=pl.BlockSpec((_N_CHAINS, _DIM), lambda t: (0, 0)),
        out_shape=jax.ShapeDtypeStruct((_N_CHAINS, _DIM), jnp.float32),
        scratch_shapes=[
            pltpu.VMEM((_N_CHAINS, _DIM), jnp.float32),
            pltpu.VMEM((_N_CHAINS, 1), jnp.float32),
        ],
        interpret=interpret,
    )


def kernel(init_state, A, loc):
    base_key = jax.random.key(42)

    def draws(t):
        key = jax.random.fold_in(base_key, t)
        k1, k2, k3 = jax.random.split(key, 3)
        log_u = jnp.log(jax.random.uniform(k2, (_N_CHAINS,), minval=1e-12))
        log_u2 = jnp.log(jax.random.uniform(k3, (_N_CHAINS,), minval=1e-12))
        return jax.random.key_data(k1), log_u, log_u2

    keys, log_u_all, log_u2_all = jax.vmap(draws)(jnp.arange(_N_STEPS))
    keys = keys.astype(jnp.uint32).reshape(_N_STEPS, 1, 2)
    lu = jnp.stack([log_u_all, log_u2_all], axis=-1)
    betas = jnp.asarray(
        _BETA_MIN ** (np.arange(_N_CHAINS) / (_N_CHAINS - 1)),
        dtype=jnp.float32).reshape(_N_CHAINS, 1)
    loc2 = loc.reshape(1, _DIM)

    return _make_pallas_call()(keys, lu, init_state, A, loc2, betas)
